# Initial kernel scaffold; baseline (speedup 1.0000x reference)
import functools

import jax
import jax.numpy as jnp
from jax import lax
from jax.experimental import pallas as pl
from jax.experimental.pallas import tpu as pltpu
from jax.experimental.pallas import tpu_sc as plsc

_N = 10000
_E = 320000
_NT = 16
_NC = 2
_B = 128
_TE = 331776
_P = 10240
_H = 8
_HH = 4


def _tc_proj_body(x_ref, w_ref, as_ref, ad_ref, h_ref, s_ref, d_ref, m_ref):
    h = jnp.dot(x_ref[...], w_ref[...], preferred_element_type=jnp.float32)
    h_ref[...] = h
    s = jnp.dot(h, as_ref[...], preferred_element_type=jnp.float32)
    d = jnp.dot(h, ad_ref[...], preferred_element_type=jnp.float32)
    s_ref[...] = s
    d_ref[...] = d
    m_ref[...] = jnp.concatenate(
        [jnp.max(s, axis=0, keepdims=True), jnp.max(d, axis=0, keepdims=True)], 0)


def _tc_proj(x, W, As, Ad):
    n, h = x.shape[0], W.shape[1]
    return pl.pallas_call(
        _tc_proj_body,
        out_shape=[
            jax.ShapeDtypeStruct((n, h), jnp.float32),
            jax.ShapeDtypeStruct((n, _H), jnp.float32),
            jax.ShapeDtypeStruct((n, _H), jnp.float32),
            jax.ShapeDtypeStruct((2, _H), jnp.float32),
        ],
    )(x, W, As, Ad)


def _tc_mid_body(acc_ref, den_ref, e_ref, b_ref, w_ref, as_ref, ad_ref,
                 h_ref, s_ref, d_ref, m_ref):
    den = jnp.dot(den_ref[...], e_ref[...], preferred_element_type=jnp.float32)
    z = acc_ref[...] / (den + 1e-16) + b_ref[...]
    z = jnp.where(z > 0.0, z, jnp.exp(z) - 1.0)
    h = jnp.dot(z, w_ref[...], preferred_element_type=jnp.float32)
    h_ref[...] = h
    s = jnp.dot(h, as_ref[...], preferred_element_type=jnp.float32)
    d = jnp.dot(h, ad_ref[...], preferred_element_type=jnp.float32)
    s_ref[...] = s
    d_ref[...] = d
    m_ref[...] = jnp.concatenate(
        [jnp.max(s, axis=0, keepdims=True), jnp.max(d, axis=0, keepdims=True)], 0)


def _tc_mid(acc, den, E, b, W, As, Ad):
    n, h = acc.shape[0], W.shape[1]
    return pl.pallas_call(
        _tc_mid_body,
        out_shape=[
            jax.ShapeDtypeStruct((n, h), jnp.float32),
            jax.ShapeDtypeStruct((n, _H), jnp.float32),
            jax.ShapeDtypeStruct((n, _H), jnp.float32),
            jax.ShapeDtypeStruct((2, _H), jnp.float32),
        ],
    )(acc, den, E, b, W, As, Ad)


def _tc_out_body(acc_ref, den_ref, e_ref, b_ref, o_ref):
    den = jnp.dot(den_ref[...], e_ref[...], preferred_element_type=jnp.float32)
    z = acc_ref[...] / (den + 1e-16) + b_ref[...]
    mx = jnp.max(z, axis=1, keepdims=True)
    zz = z - mx
    lse = jnp.log(jnp.sum(jnp.exp(zz), axis=1, keepdims=True))
    o_ref[...] = zz - lse


def _tc_out(acc, den, E, b):
    n, c = acc.shape
    return pl.pallas_call(
        _tc_out_body,
        out_shape=jax.ShapeDtypeStruct((n, c), jnp.float32),
    )(acc, den, E, b)


def _make_sc_edge(W):
    CPH = W // (_HH * 16)
    EPT = _TE // _NT
    NB = EPT // _B
    RPT = _P // _NT
    NZ = RPT // _B

    mesh = plsc.VectorSubcoreMesh(core_axis_name="c", subcore_axis_name="s")

    @functools.partial(
        pl.kernel,
        out_type=[
            jax.ShapeDtypeStruct((_NC * _P, W), jnp.float32),
            jax.ShapeDtypeStruct((_NC * _P, _HH), jnp.float32),
        ],
        mesh=mesh,
        scratch_types=[
            pltpu.VMEM((_B,), jnp.int32),
            pltpu.VMEM((_B,), jnp.int32),
            pltpu.VMEM((_B,), jnp.int32),
            pltpu.VMEM((_B,), jnp.int32),
            pltpu.VMEM((_B, _HH), jnp.float32),
            pltpu.VMEM((_B, _HH), jnp.float32),
            pltpu.VMEM((_B, _HH), jnp.float32),
            pltpu.VMEM((_B, W), jnp.float32),
            pltpu.VMEM((1, 16), jnp.float32),
            pltpu.VMEM_SHARED((_P, W), jnp.float32),
            pltpu.VMEM_SHARED((_P, _HH), jnp.float32),
            pltpu.SemaphoreType.DMA,
        ],
    )
    def sc_edge(h_t, asrc_t, adst_t, src_e, dst_e, mrow,
                acc_out, den_out,
                sidx, didx, rs, rd, aS, aD, ee, hrows, mv, acc_sh, den_sh, sem):
        c = lax.axis_index("c")
        t = lax.axis_index("s")
        zero16 = jnp.zeros((16,), jnp.float32)
        i16 = lax.iota(jnp.int32, 16)
        r0 = lax.shift_right_logical(i16, 2)
        c0 = lax.bitwise_and(i16, 3)

        pltpu.sync_copy(mrow.at[pl.ds(c, 1)], mv)
        mreg = mv[0, :]

        def zh_body(e, _):
            for kk in range(W // 16):
                hrows[e, pl.ds(kk * 16, 16)] = zero16
            return 0
        lax.fori_loop(0, _B, zh_body, 0)

        def zee_body(g, _):
            plsc.store_scatter(ee, [r0 + 4 * g, c0], zero16)
            return 0
        lax.fori_loop(0, _B * _HH // 16, zee_body, 0)

        def zacc_body(q, _):
            roff = t * RPT + q * _B
            pltpu.sync_copy(hrows, acc_sh.at[pl.ds(roff, _B)])
            pltpu.sync_copy(ee, den_sh.at[pl.ds(roff, _B)])
            return 0
        lax.fori_loop(0, NZ, zacc_body, 0)
        plsc.subcore_barrier()

        def idx_body(g, _):
            s16 = sidx[pl.ds(g * 16, 16)]
            rs[pl.ds(g * 16, 16)] = s16 * 2 + c
            d16 = didx[pl.ds(g * 16, 16)]
            rd[pl.ds(g * 16, 16)] = d16 * 2 + c
            return 0

        def ee_body(g, _):
            rows = r0 + 4 * g
            va = plsc.load_gather(aS, [rows, c0])
            vb = plsc.load_gather(aD, [rows, c0])
            ev = va + vb
            ev = jnp.where(ev > 0.0, ev, 0.2 * ev)
            ev = jnp.exp(ev - mreg)
            plsc.store_scatter(ee, [rows, c0], ev)
            return 0

        def mul_body(e, _):
            er = jnp.full((16,), e, dtype=jnp.int32)
            for j in range(_HH):
                jr = jnp.full((16,), j, dtype=jnp.int32)
                w16 = plsc.load_gather(ee, [er, jr])
                for k in range(CPH):
                    col = (j * CPH + k) * 16
                    hrows[e, pl.ds(col, 16)] = hrows[e, pl.ds(col, 16)] * w16
            return 0

        def batch_body(b, _):
            base = t * EPT + b * _B
            pltpu.sync_copy(src_e.at[pl.ds(base, _B)], sidx)
            pltpu.sync_copy(dst_e.at[pl.ds(base, _B)], didx)
            lax.fori_loop(0, _B // 16, idx_body, 0)
            pltpu.async_copy(asrc_t.at[rs], aS, sem).wait()
            pltpu.async_copy(adst_t.at[rd], aD, sem).wait()
            lax.fori_loop(0, _B * _HH // 16, ee_body, 0)
            pltpu.sync_copy(ee, den_sh.at[didx], add=True)
            pltpu.async_copy(h_t.at[rs], hrows, sem).wait()
            lax.fori_loop(0, _B, mul_body, 0)
            pltpu.sync_copy(hrows, acc_sh.at[didx], add=True)
            return 0
        lax.fori_loop(0, NB, batch_body, 0)
        plsc.subcore_barrier()

        def out_body(q, _):
            roff = t * RPT + q * _B
            pltpu.sync_copy(acc_sh.at[pl.ds(roff, _B)],
                            acc_out.at[pl.ds(c * _P + roff, _B)])
            pltpu.sync_copy(den_sh.at[pl.ds(roff, _B)],
                            den_out.at[pl.ds(c * _P + roff, _B)])
            return 0
        lax.fori_loop(0, NZ, out_body, 0)

    return sc_edge


_sc_edge_64 = _make_sc_edge(64)
_sc_edge_128 = _make_sc_edge(128)


def _attn_mats(a, H, C):
    return (jnp.eye(H, dtype=jnp.float32)[:, None, :] *
            a[0][:, :, None]).reshape(H * C, H)


def _expand_mat(H, C):
    return jnp.kron(jnp.eye(H, dtype=jnp.float32),
                    jnp.ones((1, C), dtype=jnp.float32))


def _tables(h, s, d, K):
    hp = jnp.zeros((_P, K), jnp.float32).at[:_N].set(h)
    sp = jnp.zeros((_P, _H), jnp.float32).at[:_N].set(s)
    dp = jnp.zeros((_P, _H), jnp.float32).at[:_N].set(d)
    return (hp.reshape(_NC * _P, K // _NC),
            sp.reshape(_NC * _P, _HH),
            dp.reshape(_NC * _P, _HH))


def _mrow(msd):
    m = msd[0] + msd[1]
    m = jnp.where(m > 0.0, m, 0.2 * m)
    return jnp.tile(m.reshape(_NC, _HH), (1, 4))


def _assemble(acc_raw, den_raw, W):
    acc = acc_raw.reshape(_P, _NC, W)[:_N].transpose(0, 1, 2)
    den = den_raw.reshape(_P, _NC, _HH)[:_N]
    return acc.reshape(_N, _NC * W), den.reshape(_N, _H)


def kernel(x, edge_index, W1, a_src1, a_dst1, b1, W2, a_src2, a_dst2, b2):
    src = edge_index[0].astype(jnp.int32)
    dst = edge_index[1].astype(jnp.int32)
    loop = jnp.arange(_N, dtype=jnp.int32)
    pad = _TE - (_E + _N)
    padv = jnp.full((pad,), _N, jnp.int32)
    src_p = jnp.concatenate([src, loop, padv])
    dst_p = jnp.concatenate([dst, loop, padv])

    h1, s1, d1, m1 = _tc_proj(x, W1, _attn_mats(a_src1, _H, 16),
                              _attn_mats(a_dst1, _H, 16))
    h1_t, s1_t, d1_t = _tables(h1, s1, d1, 128)
    acc1_raw, den1_raw = _sc_edge_64(h1_t, s1_t, d1_t, src_p, dst_p, _mrow(m1))
    acc1, den1 = _assemble(acc1_raw, den1_raw, 64)

    h2, s2, d2, m2 = _tc_mid(acc1, den1, _expand_mat(_H, 16),
                             b1.reshape(1, 128), W2,
                             _attn_mats(a_src2, _H, 32),
                             _attn_mats(a_dst2, _H, 32))
    h2_t, s2_t, d2_t = _tables(h2, s2, d2, 256)
    acc2_raw, den2_raw = _sc_edge_128(h2_t, s2_t, d2_t, src_p, dst_p, _mrow(m2))
    acc2, den2 = _assemble(acc2_raw, den2_raw, 128)

    return _tc_out(acc2, den2, _expand_mat(_H, 32), b2.reshape(1, 256))


# speedup vs baseline: 29.5581x; 29.5581x over previous
import functools

import jax
import jax.numpy as jnp
from jax import lax
from jax.experimental import pallas as pl
from jax.experimental.pallas import tpu as pltpu
from jax.experimental.pallas import tpu_sc as plsc

_N = 10000
_E = 320000
_NT = 16
_NC = 2
_B = 128
_TE = 331776
_P = 10240
_H = 8
_HH = 4


def _tc_proj_body(x_ref, w_ref, as_ref, ad_ref, h_ref, s_ref, d_ref, m_ref):
    h = jnp.dot(x_ref[...], w_ref[...], preferred_element_type=jnp.float32)
    h_ref[...] = h
    s = jnp.dot(h, as_ref[...], preferred_element_type=jnp.float32)
    d = jnp.dot(h, ad_ref[...], preferred_element_type=jnp.float32)
    s_ref[...] = s
    d_ref[...] = d
    m_ref[...] = jnp.concatenate(
        [jnp.max(s, axis=0, keepdims=True), jnp.max(d, axis=0, keepdims=True)], 0)


def _tc_proj(x, W, As, Ad):
    n, h = x.shape[0], W.shape[1]
    return pl.pallas_call(
        _tc_proj_body,
        out_shape=[
            jax.ShapeDtypeStruct((n, h), jnp.float32),
            jax.ShapeDtypeStruct((n, _H), jnp.float32),
            jax.ShapeDtypeStruct((n, _H), jnp.float32),
            jax.ShapeDtypeStruct((2, _H), jnp.float32),
        ],
    )(x, W, As, Ad)


def _tc_mid_body(acc_ref, den_ref, e_ref, b_ref, w_ref, as_ref, ad_ref,
                 h_ref, s_ref, d_ref, m_ref):
    den = jnp.dot(den_ref[...], e_ref[...], preferred_element_type=jnp.float32)
    z = acc_ref[...] / (den + 1e-16) + b_ref[...]
    z = jnp.where(z > 0.0, z, jnp.exp(z) - 1.0)
    h = jnp.dot(z, w_ref[...], preferred_element_type=jnp.float32)
    h_ref[...] = h
    s = jnp.dot(h, as_ref[...], preferred_element_type=jnp.float32)
    d = jnp.dot(h, ad_ref[...], preferred_element_type=jnp.float32)
    s_ref[...] = s
    d_ref[...] = d
    m_ref[...] = jnp.concatenate(
        [jnp.max(s, axis=0, keepdims=True), jnp.max(d, axis=0, keepdims=True)], 0)


def _tc_mid(acc, den, E, b, W, As, Ad):
    n, h = acc.shape[0], W.shape[1]
    return pl.pallas_call(
        _tc_mid_body,
        out_shape=[
            jax.ShapeDtypeStruct((n, h), jnp.float32),
            jax.ShapeDtypeStruct((n, _H), jnp.float32),
            jax.ShapeDtypeStruct((n, _H), jnp.float32),
            jax.ShapeDtypeStruct((2, _H), jnp.float32),
        ],
    )(acc, den, E, b, W, As, Ad)


def _tc_out_body(acc_ref, den_ref, e_ref, b_ref, o_ref):
    den = jnp.dot(den_ref[...], e_ref[...], preferred_element_type=jnp.float32)
    z = acc_ref[...] / (den + 1e-16) + b_ref[...]
    mx = jnp.max(z, axis=1, keepdims=True)
    zz = z - mx
    lse = jnp.log(jnp.sum(jnp.exp(zz), axis=1, keepdims=True))
    o_ref[...] = zz - lse


def _tc_out(acc, den, E, b):
    n, c = acc.shape
    return pl.pallas_call(
        _tc_out_body,
        out_shape=jax.ShapeDtypeStruct((n, c), jnp.float32),
    )(acc, den, E, b)


@functools.lru_cache(maxsize=None)
def _make_sc_edge(W):
    CPH = W // (_HH * 16)
    EPT = _TE // _NT
    NB = EPT // _B
    RPT = _P // _NT
    NZ = RPT // _B

    mesh = plsc.VectorSubcoreMesh(core_axis_name="c", subcore_axis_name="s")

    @functools.partial(
        pl.kernel,
        out_type=[
            jax.ShapeDtypeStruct((_NC * _P, W), jnp.float32),
            jax.ShapeDtypeStruct((_NC * _P, 16), jnp.float32),
        ],
        mesh=mesh,
        compiler_params=pltpu.CompilerParams(use_tc_tiling_on_sc=False),
        scratch_types=[
            pltpu.VMEM((_B,), jnp.int32),
            pltpu.VMEM((_B,), jnp.int32),
            pltpu.VMEM((_B,), jnp.int32),
            pltpu.VMEM((_B,), jnp.int32),
            pltpu.VMEM((_B, 16), jnp.float32),
            pltpu.VMEM((_B, 16), jnp.float32),
            pltpu.VMEM((_B, 16), jnp.float32),
            pltpu.VMEM((_B, W), jnp.float32),
            pltpu.VMEM((1, 16), jnp.float32),
            pltpu.VMEM_SHARED((_P, W), jnp.float32),
            pltpu.VMEM_SHARED((_P, 16), jnp.float32),
            pltpu.SemaphoreType.DMA,
        ],
    )
    def sc_edge(h_t, asrc_t, adst_t, src_e, dst_e, mrow,
                acc_out, den_out,
                sidx, didx, rs, rd, aS, aD, eer, hrows, mv,
                acc_sh, den_sh, sem):
        c = lax.axis_index("c")
        t = lax.axis_index("s")
        zero16 = jnp.zeros((16,), jnp.float32)

        pltpu.sync_copy(mrow.at[pl.ds(c, 1)], mv)
        mreg = mv[0, :]

        def zh_body(e, _):
            for kk in range(W // 16):
                hrows[e, pl.ds(kk * 16, 16)] = zero16
            eer[e, pl.ds(0, 16)] = zero16
            return 0
        lax.fori_loop(0, _B, zh_body, 0)

        def zacc_body(q, _):
            roff = t * RPT + q * _B
            pltpu.sync_copy(hrows, acc_sh.at[pl.ds(roff, _B)])
            pltpu.sync_copy(eer, den_sh.at[pl.ds(roff, _B)])
            return 0
        lax.fori_loop(0, NZ, zacc_body, 0)
        plsc.subcore_barrier()

        def idx_body(g, _):
            s16 = sidx[pl.ds(g * 16, 16)]
            rs[pl.ds(g * 16, 16)] = s16 * 2 + c
            d16 = didx[pl.ds(g * 16, 16)]
            rd[pl.ds(g * 16, 16)] = d16 * 2 + c
            return 0

        def mul_body(e, _):
            va = aS[e, pl.ds(0, 16)]
            vb = aD[e, pl.ds(0, 16)]
            ev = va + vb
            ev = jnp.where(ev > 0.0, ev, 0.2 * ev)
            ev = jnp.exp(ev - mreg)
            eer[e, pl.ds(0, 16)] = ev
            for j in range(_HH):
                w16 = jnp.broadcast_to(ev[j], (16,))
                for k in range(CPH):
                    col = (j * CPH + k) * 16
                    hrows[e, pl.ds(col, 16)] = hrows[e, pl.ds(col, 16)] * w16
            return 0

        def batch_body(b, _):
            base = t * EPT + b * _B
            pltpu.sync_copy(src_e.at[pl.ds(base, _B)], sidx)
            pltpu.sync_copy(dst_e.at[pl.ds(base, _B)], didx)
            lax.fori_loop(0, _B // 16, idx_body, 0)
            pltpu.async_copy(asrc_t.at[rs], aS, sem).wait()
            pltpu.async_copy(adst_t.at[rd], aD, sem).wait()
            pltpu.async_copy(h_t.at[rs], hrows, sem).wait()
            lax.fori_loop(0, _B, mul_body, 0)
            pltpu.sync_copy(eer, den_sh.at[didx], add=True)
            pltpu.sync_copy(hrows, acc_sh.at[didx], add=True)
            return 0
        lax.fori_loop(0, NB, batch_body, 0)
        plsc.subcore_barrier()

        def out_body(q, _):
            roff = t * RPT + q * _B
            pltpu.sync_copy(acc_sh.at[pl.ds(roff, _B)],
                            acc_out.at[pl.ds(c * _P + roff, _B)])
            pltpu.sync_copy(den_sh.at[pl.ds(roff, _B)],
                            den_out.at[pl.ds(c * _P + roff, _B)])
            return 0
        lax.fori_loop(0, NZ, out_body, 0)

    return sc_edge


def _attn_mats(a, H, C):
    return (jnp.eye(H, dtype=jnp.float32)[:, None, :] *
            a[0][:, :, None]).reshape(H * C, H)


def _expand_mat(H, C):
    return jnp.kron(jnp.eye(H, dtype=jnp.float32),
                    jnp.ones((1, C), dtype=jnp.float32))


def _tables(h, s, d, K):
    hp = jnp.zeros((_P, K), jnp.float32).at[:_N].set(h)
    sp = jnp.zeros((_P, _H), jnp.float32).at[:_N].set(s)
    dp = jnp.zeros((_P, _H), jnp.float32).at[:_N].set(d)
    tile4 = lambda a: jnp.tile(a.reshape(_P, _NC, _HH),
                               (1, 1, 4)).reshape(_NC * _P, 16)
    return hp.reshape(_NC * _P, K // _NC), tile4(sp), tile4(dp)


def _mrow(msd):
    m = msd[0] + msd[1]
    m = jnp.where(m > 0.0, m, 0.2 * m)
    return jnp.tile(m.reshape(_NC, _HH), (1, 4))


def _assemble(acc_raw, den_raw, W):
    acc = acc_raw.reshape(_NC, _P, W)[:, :_N].transpose(1, 0, 2)
    den = den_raw.reshape(_NC, _P, 16)[:, :_N, :_HH].transpose(1, 0, 2)
    return acc.reshape(_N, _NC * W), den.reshape(_N, _H)


def kernel(x, edge_index, W1, a_src1, a_dst1, b1, W2, a_src2, a_dst2, b2):
    src = edge_index[0].astype(jnp.int32)
    dst = edge_index[1].astype(jnp.int32)
    loop = jnp.arange(_N, dtype=jnp.int32)
    pad = _TE - (_E + _N)
    padv = jnp.full((pad,), _N, jnp.int32)
    src_p = jnp.concatenate([src, loop, padv])
    dst_p = jnp.concatenate([dst, loop, padv])

    h1, s1, d1, m1 = _tc_proj(x, W1, _attn_mats(a_src1, _H, 16),
                              _attn_mats(a_dst1, _H, 16))
    h1_t, s1_t, d1_t = _tables(h1, s1, d1, 128)
    acc1_raw, den1_raw = _make_sc_edge(64)(h1_t, s1_t, d1_t, src_p, dst_p,
                                           _mrow(m1))
    acc1, den1 = _assemble(acc1_raw, den1_raw, 64)

    h2, s2, d2, m2 = _tc_mid(acc1, den1, _expand_mat(_H, 16),
                             b1.reshape(1, 128), W2,
                             _attn_mats(a_src2, _H, 32),
                             _attn_mats(a_dst2, _H, 32))
    h2_t, s2_t, d2_t = _tables(h2, s2, d2, 256)
    acc2_raw, den2_raw = _make_sc_edge(128)(h2_t, s2_t, d2_t, src_p, dst_p,
                                            _mrow(m2))
    acc2, den2 = _assemble(acc2_raw, den2_raw, 128)

    return _tc_out(acc2, den2, _expand_mat(_H, 32), b2.reshape(1, 256))


# speedup vs baseline: 31.1373x; 1.0534x over previous
import functools

import jax
import jax.numpy as jnp
from jax import lax
from jax.experimental import pallas as pl
from jax.experimental.pallas import tpu as pltpu
from jax.experimental.pallas import tpu_sc as plsc

_N = 10000
_E = 320000
_NT = 16
_NC = 2
_B = 128
_TE = 331776
_P = 10240
_H = 8
_HH = 4


def _tc_proj_body(x_ref, w_ref, as_ref, ad_ref, h_ref, s_ref, d_ref, m_ref):
    h = jnp.dot(x_ref[...], w_ref[...], preferred_element_type=jnp.float32)
    h_ref[...] = h
    s = jnp.dot(h, as_ref[...], preferred_element_type=jnp.float32)
    d = jnp.dot(h, ad_ref[...], preferred_element_type=jnp.float32)
    s_ref[...] = s
    d_ref[...] = d
    m_ref[...] = jnp.concatenate(
        [jnp.max(s, axis=0, keepdims=True), jnp.max(d, axis=0, keepdims=True)], 0)


def _tc_proj(x, W, As, Ad):
    n, h = x.shape[0], W.shape[1]
    return pl.pallas_call(
        _tc_proj_body,
        out_shape=[
            jax.ShapeDtypeStruct((n, h), jnp.float32),
            jax.ShapeDtypeStruct((n, _H), jnp.float32),
            jax.ShapeDtypeStruct((n, _H), jnp.float32),
            jax.ShapeDtypeStruct((2, _H), jnp.float32),
        ],
    )(x, W, As, Ad)


def _tc_mid_body(acc_ref, den_ref, e_ref, b_ref, w_ref, as_ref, ad_ref,
                 h_ref, s_ref, d_ref, m_ref):
    den = jnp.dot(den_ref[...], e_ref[...], preferred_element_type=jnp.float32)
    z = acc_ref[...] / (den + 1e-16) + b_ref[...]
    z = jnp.where(z > 0.0, z, jnp.exp(z) - 1.0)
    h = jnp.dot(z, w_ref[...], preferred_element_type=jnp.float32)
    h_ref[...] = h
    s = jnp.dot(h, as_ref[...], preferred_element_type=jnp.float32)
    d = jnp.dot(h, ad_ref[...], preferred_element_type=jnp.float32)
    s_ref[...] = s
    d_ref[...] = d
    m_ref[...] = jnp.concatenate(
        [jnp.max(s, axis=0, keepdims=True), jnp.max(d, axis=0, keepdims=True)], 0)


def _tc_mid(acc, den, E, b, W, As, Ad):
    n, h = acc.shape[0], W.shape[1]
    return pl.pallas_call(
        _tc_mid_body,
        out_shape=[
            jax.ShapeDtypeStruct((n, h), jnp.float32),
            jax.ShapeDtypeStruct((n, _H), jnp.float32),
            jax.ShapeDtypeStruct((n, _H), jnp.float32),
            jax.ShapeDtypeStruct((2, _H), jnp.float32),
        ],
    )(acc, den, E, b, W, As, Ad)


def _tc_out_body(acc_ref, den_ref, e_ref, b_ref, o_ref):
    den = jnp.dot(den_ref[...], e_ref[...], preferred_element_type=jnp.float32)
    z = acc_ref[...] / (den + 1e-16) + b_ref[...]
    mx = jnp.max(z, axis=1, keepdims=True)
    zz = z - mx
    lse = jnp.log(jnp.sum(jnp.exp(zz), axis=1, keepdims=True))
    o_ref[...] = zz - lse


def _tc_out(acc, den, E, b):
    n, c = acc.shape
    return pl.pallas_call(
        _tc_out_body,
        out_shape=jax.ShapeDtypeStruct((n, c), jnp.float32),
    )(acc, den, E, b)


@functools.lru_cache(maxsize=None)
def _make_sc_edge(W, split, call):
    NP = _NC * split
    HPP = _H // NP
    CPH = W // (HPP * 16)
    EPT = _TE // _NT
    NB = EPT // _B
    RPT = _P // _NT
    NZ = RPT // _B

    mesh = plsc.VectorSubcoreMesh(core_axis_name="c", subcore_axis_name="s")

    @functools.partial(
        pl.kernel,
        out_type=[
            jax.ShapeDtypeStruct((_NC * _P, W), jnp.float32),
            jax.ShapeDtypeStruct((_NC * _P, 16), jnp.float32),
        ],
        mesh=mesh,
        compiler_params=pltpu.CompilerParams(use_tc_tiling_on_sc=False),
        scratch_types=[
            pltpu.VMEM((_B,), jnp.int32),
            [pltpu.VMEM((_B,), jnp.int32)] * 2,
            [pltpu.VMEM((_B,), jnp.int32)] * 2,
            [pltpu.VMEM((_B,), jnp.int32)] * 2,
            [pltpu.VMEM((_B, 16), jnp.float32)] * 2,
            [pltpu.VMEM((_B, 16), jnp.float32)] * 2,
            [pltpu.VMEM((_B, 16), jnp.float32)] * 2,
            [pltpu.VMEM((_B, W), jnp.float32)] * 2,
            pltpu.VMEM((1, 16), jnp.float32),
            pltpu.VMEM_SHARED((_P, W), jnp.float32),
            pltpu.VMEM_SHARED((_P, 16), jnp.float32),
            [pltpu.SemaphoreType.DMA] * 2,
            [pltpu.SemaphoreType.DMA] * 2,
        ],
    )
    def sc_edge(h_t, asrc_t, adst_t, src_e, dst_e, mrow,
                acc_out, den_out,
                sidx, didx, rs, rd, aS, aD, eer, hrows, mv,
                acc_sh, den_sh, gsem, ssem):
        c = lax.axis_index("c")
        t = lax.axis_index("s")
        part = c * split + call
        zero16 = jnp.zeros((16,), jnp.float32)
        padn16 = jnp.full((16,), _N, jnp.int32)

        pltpu.sync_copy(mrow.at[pl.ds(c, 1)], mv)
        mreg = mv[0, :]

        def zh_body(e, _):
            for sl in range(2):
                for kk in range(W // 16):
                    hrows[sl][e, pl.ds(kk * 16, 16)] = zero16
                eer[sl][e, pl.ds(0, 16)] = zero16
            return 0
        lax.fori_loop(0, _B, zh_body, 0)

        def zacc_body(q, _):
            roff = t * RPT + q * _B
            pltpu.sync_copy(hrows[0], acc_sh.at[pl.ds(roff, _B)])
            pltpu.sync_copy(eer[0], den_sh.at[pl.ds(roff, _B)])
            return 0
        lax.fori_loop(0, NZ, zacc_body, 0)
        plsc.subcore_barrier()

        def load_and_transform(p, sl):
            base = t * EPT + p * _B
            pltpu.sync_copy(src_e.at[pl.ds(base, _B)], sidx)
            for g in range(_B // 16):
                s16 = sidx[pl.ds(g * 16, 16)]
                rs[sl][pl.ds(g * 16, 16)] = s16 * NP + part
            pltpu.sync_copy(dst_e.at[pl.ds(base, _B)], sidx)
            for g in range(_B // 16):
                d16 = sidx[pl.ds(g * 16, 16)]
                didx[sl][pl.ds(g * 16, 16)] = d16
                rd[sl][pl.ds(g * 16, 16)] = d16 * NP + part
            pltpu.async_copy(asrc_t.at[rs[sl]], aS[sl], gsem[sl])
            pltpu.async_copy(adst_t.at[rd[sl]], aD[sl], gsem[sl])
            pltpu.async_copy(h_t.at[rs[sl]], hrows[sl], gsem[sl])

        def wait_gathers(sl):
            pltpu.make_async_copy(asrc_t.at[rs[sl]], aS[sl], gsem[sl]).wait()
            pltpu.make_async_copy(adst_t.at[rd[sl]], aD[sl], gsem[sl]).wait()
            pltpu.make_async_copy(h_t.at[rs[sl]], hrows[sl], gsem[sl]).wait()

        def wait_scatters(sl):
            pltpu.make_async_copy(eer[sl], den_sh.at[didx[sl]],
                                  ssem[sl]).wait()
            pltpu.make_async_copy(hrows[sl], acc_sh.at[didx[sl]],
                                  ssem[sl]).wait()

        def mul_and_scatter(sl):
            def mul_body(e, _):
                va = aS[sl][e, pl.ds(0, 16)]
                vb = aD[sl][e, pl.ds(0, 16)]
                ev = va + vb
                ev = jnp.where(ev > 0.0, ev, 0.2 * ev)
                ev = jnp.exp(ev - mreg)
                eer[sl][e, pl.ds(0, 16)] = ev
                for j in range(HPP):
                    w16 = jnp.broadcast_to(ev[j], (16,))
                    for k in range(CPH):
                        col = (j * CPH + k) * 16
                        hrows[sl][e, pl.ds(col, 16)] = (
                            hrows[sl][e, pl.ds(col, 16)] * w16)
                return 0
            lax.fori_loop(0, _B, mul_body, 0, unroll=2)
            pltpu.async_copy(eer[sl], den_sh.at[didx[sl]], ssem[sl], add=True)
            pltpu.async_copy(hrows[sl], acc_sh.at[didx[sl]], ssem[sl],
                             add=True)

        for g in range(_B // 16):
            didx[1][pl.ds(g * 16, 16)] = padn16
        pltpu.async_copy(eer[1], den_sh.at[didx[1]], ssem[1], add=True)
        pltpu.async_copy(hrows[1], acc_sh.at[didx[1]], ssem[1], add=True)
        load_and_transform(0, 0)

        def pipe_body(gi, _):
            for s in range(2):
                b = gi * 2 + s
                sl, nsl = s, 1 - s
                wait_scatters(nsl)
                load_and_transform(b + 1, nsl)
                wait_gathers(sl)
                mul_and_scatter(sl)
            return 0
        lax.fori_loop(0, NB // 2, pipe_body, 0)
        wait_scatters(1)
        wait_gathers(0)
        plsc.subcore_barrier()

        def out_body(q, _):
            roff = t * RPT + q * _B
            pltpu.sync_copy(acc_sh.at[pl.ds(roff, _B)],
                            acc_out.at[pl.ds(c * _P + roff, _B)])
            pltpu.sync_copy(den_sh.at[pl.ds(roff, _B)],
                            den_out.at[pl.ds(c * _P + roff, _B)])
            return 0
        lax.fori_loop(0, NZ, out_body, 0)

    return sc_edge


def _attn_mats(a, H, C):
    return (jnp.eye(H, dtype=jnp.float32)[:, None, :] *
            a[0][:, :, None]).reshape(H * C, H)


def _expand_mat(H, C):
    return jnp.kron(jnp.eye(H, dtype=jnp.float32),
                    jnp.ones((1, C), dtype=jnp.float32))


def _tables(h, s, d, K, split):
    NP = _NC * split
    HPP = _H // NP
    hp = jnp.zeros((_P, K), jnp.float32).at[:_N].set(h)
    sp = jnp.zeros((_P, _H), jnp.float32).at[:_N].set(s)
    dp = jnp.zeros((_P, _H), jnp.float32).at[:_N].set(d)
    tile16 = lambda a: jnp.tile(a.reshape(_P, NP, HPP),
                                (1, 1, 16 // HPP)).reshape(NP * _P, 16)
    return hp.reshape(NP * _P, K // NP), tile16(sp), tile16(dp)


def _mrow(msd, split, call):
    NP = _NC * split
    HPP = _H // NP
    m = msd[0] + msd[1]
    m = jnp.where(m > 0.0, m, 0.2 * m)
    mq = m.reshape(NP, HPP)[jnp.array([call, split + call])]
    return jnp.tile(mq, (1, 16 // HPP))


def _sc_layer(h, s, d, m, K, split, src_p, dst_p):
    NP = _NC * split
    HPP = _H // NP
    W = K // NP
    h_t, s_t, d_t = _tables(h, s, d, K, split)
    accs, dens = [None] * NP, [None] * NP
    for call in range(split):
        acc_raw, den_raw = _make_sc_edge(W, split, call)(
            h_t, s_t, d_t, src_p, dst_p, _mrow(m, split, call))
        acc = acc_raw.reshape(_NC, _P, W)[:, :_N]
        den = den_raw.reshape(_NC, _P, 16)[:, :_N, :HPP]
        for c in range(_NC):
            accs[c * split + call] = acc[c]
            dens[c * split + call] = den[c]
    return (jnp.concatenate(accs, axis=1),
            jnp.concatenate(dens, axis=1))


def kernel(x, edge_index, W1, a_src1, a_dst1, b1, W2, a_src2, a_dst2, b2):
    src = edge_index[0].astype(jnp.int32)
    dst = edge_index[1].astype(jnp.int32)
    loop = jnp.arange(_N, dtype=jnp.int32)
    pad = _TE + _B - (_E + _N)
    padv = jnp.full((pad,), _N, jnp.int32)
    src_p = jnp.concatenate([src, loop, padv])
    dst_p = jnp.concatenate([dst, loop, padv])

    h1, s1, d1, m1 = _tc_proj(x, W1, _attn_mats(a_src1, _H, 16),
                              _attn_mats(a_dst1, _H, 16))
    acc1, den1 = _sc_layer(h1, s1, d1, m1, 128, 1, src_p, dst_p)

    h2, s2, d2, m2 = _tc_mid(acc1, den1, _expand_mat(_H, 16),
                             b1.reshape(1, 128), W2,
                             _attn_mats(a_src2, _H, 32),
                             _attn_mats(a_dst2, _H, 32))
    acc2, den2 = _sc_layer(h2, s2, d2, m2, 256, 2, src_p, dst_p)

    return _tc_out(acc2, den2, _expand_mat(_H, 32), b2.reshape(1, 256))
